# Initial kernel scaffold; baseline (speedup 1.0000x reference)
import jax
import jax.numpy as jnp
from jax.experimental import pallas as pl


def kernel(x, edge_index, nodes, W_proj, b_proj, W_out, b_out):
    raise NotImplementedError("write your pallas kernel here")


# speedup vs baseline: 1.4962x; 1.4962x over previous
import functools

import jax
import jax.numpy as jnp
from jax import lax
from jax.experimental import pallas as pl
from jax.experimental.pallas import tpu as pltpu
from jax.experimental.pallas import tpu_sc as plsc

TAU = 2.0
V_TH = 1.0

NC = 2
NS = 16
NQ = 4
LK = 128
GC = 8
NB = 1024
NBA = 2000


def _tc_proj_kernel(x_ref, wp_ref, bp_ref, out_ref):
    out_ref[...] = (jnp.dot(x_ref[0], wp_ref[...],
                            preferred_element_type=jnp.float32)
                    + bp_ref[...])[None]


def _sc_y_kernel(T, NPAD, NG, table_h, gidx_h, didx_h, zbig_h, yout_h,
                 gall_v, dall_v, rows0_v, rows1_v, y_acc,
                 gsem0, gsem1, ssem0, ssem1):
    cid = lax.axis_index("c")
    sid = lax.axis_index("s")
    rpt = NPAD // NS
    rows_b = (rows0_v, rows1_v)
    gsem_b = (gsem0, gsem1)
    ssem_b = (ssem0, ssem1)
    for t in range(T):
        for p in range(NQ // NC):
            q = cid * (NQ // NC) + p
            pltpu.sync_copy(zbig_h, y_acc.at[pl.ds(sid * rpt, rpt)])
            plsc.subcore_barrier()

            def group(g, carry):
                pltpu.sync_copy(gidx_h.at[q, t, sid, g], gall_v)
                pltpu.sync_copy(didx_h.at[t, sid, g], dall_v)
                descs = [None] * GC
                gds = [pltpu.async_copy(table_h.at[gall_v.at[0]], rows0_v,
                                        gsem0)] + [None] * (GC - 1)
                for k in range(GC):
                    b = k % 2
                    b2 = (k + 1) % 2
                    if k + 1 < GC:
                        if k >= 1:
                            descs[k - 1].wait()
                        gds[k + 1] = pltpu.async_copy(
                            table_h.at[gall_v.at[k + 1]], rows_b[b2],
                            gsem_b[b2])
                    gds[k].wait()
                    descs[k] = pltpu.async_copy(
                        rows_b[b], y_acc.at[dall_v.at[k]], ssem_b[b],
                        add=True)
                descs[GC - 2].wait()
                descs[GC - 1].wait()
                return carry

            lax.fori_loop(0, NG, group, 0)
            plsc.subcore_barrier()
            rows = pl.ds(sid * rpt, rpt)
            pltpu.sync_copy(y_acc.at[rows], yout_h.at[t, q, rows])


def _tc_lif_kernel(T, y_ref, wo_ref, bo_ref, out_ref, v_ref, acc_ref):
    t = pl.program_id(1)

    @pl.when(t == 0)
    def _():
        v_ref[...] = jnp.zeros_like(v_ref)
        acc_ref[...] = jnp.broadcast_to(bo_ref[...], acc_ref.shape)

    inp = jnp.concatenate([y_ref[0, q] for q in range(NQ)], axis=-1)
    v = v_ref[...] + (inp - v_ref[...]) * (1.0 / TAU)
    s = (v >= V_TH).astype(jnp.float32)
    v_ref[...] = v * (1.0 - s)
    acc_ref[...] = acc_ref[...] + jnp.dot(s, wo_ref[0],
                                          preferred_element_type=jnp.float32)

    @pl.when(t == T - 1)
    def _():
        out_ref[...] = acc_ref[...]


def kernel(x, edge_index, nodes, W_proj, b_proj, W_out, b_out):
    T, N, DIN = x.shape
    E = edge_index.shape[2]
    DH = W_proj.shape[1]
    DOUT = W_out.shape[1]
    NPAD = ((N + NB - 1) // NB) * NB
    EPT = E // NS
    NG = (EPT + GC * LK - 1) // (GC * LK)
    CH = NG * GC

    bp = b_proj.reshape(1, DH)
    embd = pl.pallas_call(
        _tc_proj_kernel,
        grid=(T, N // NBA),
        in_specs=[
            pl.BlockSpec((1, NBA, DIN), lambda t, b: (t, b, 0)),
            pl.BlockSpec((DIN, DH), lambda t, b: (0, 0)),
            pl.BlockSpec((1, DH), lambda t, b: (0, 0)),
        ],
        out_specs=pl.BlockSpec((1, NBA, DH), lambda t, b: (t, b, 0)),
        out_shape=jax.ShapeDtypeStruct((T, N, DH), jnp.float32),
    )(x, W_proj, bp)
    table = embd.reshape(T * N * NQ, DH // NQ)

    src = edge_index[:, 0, :].reshape(T, NS, EPT)
    dst = edge_index[:, 1, :].reshape(T, NS, EPT)
    tbase = (jnp.arange(T, dtype=jnp.int32) * (NQ * N))[None, :, None, None]
    qoff = jnp.arange(NQ, dtype=jnp.int32)[:, None, None, None]
    gidx = tbase + qoff + NQ * src[None]
    pad = CH * LK - EPT
    gidx = jnp.pad(gidx, ((0, 0), (0, 0), (0, 0), (0, pad)))
    gidx = gidx.reshape(NQ, T, NS, NG, GC, LK)
    didx = jnp.pad(dst, ((0, 0), (0, 0), (0, pad)), constant_values=N)
    didx = didx.reshape(T, NS, NG, GC, LK)
    zbig = jnp.zeros((NPAD // NS, 128), jnp.float32)

    mesh = plsc.VectorSubcoreMesh(core_axis_name="c", subcore_axis_name="s")
    y_fn = functools.partial(
        pl.kernel,
        out_type=jax.ShapeDtypeStruct((T, NQ, NPAD, 128), jnp.float32),
        mesh=mesh,
        scratch_types=(
            pltpu.VMEM((GC, LK), jnp.int32),
            pltpu.VMEM((GC, LK), jnp.int32),
            pltpu.VMEM((LK, 128), jnp.float32),
            pltpu.VMEM((LK, 128), jnp.float32),
            pltpu.VMEM_SHARED((NPAD, 128), jnp.float32),
            pltpu.SemaphoreType.DMA,
            pltpu.SemaphoreType.DMA,
            pltpu.SemaphoreType.DMA,
            pltpu.SemaphoreType.DMA,
        ),
    )(functools.partial(_sc_y_kernel, T, NPAD, NG))
    yout = y_fn(table, gidx, didx, zbig)

    wo = jnp.pad(W_out.reshape(T, DH, DOUT),
                 ((0, 0), (0, 0), (0, 128 - DOUT)))
    bo = jnp.pad(b_out, (0, 128 - DOUT)).reshape(1, 128)

    out = pl.pallas_call(
        functools.partial(_tc_lif_kernel, T),
        grid=(NPAD // NB, T),
        in_specs=[
            pl.BlockSpec((1, NQ, NB, 128), lambda b, t: (t, 0, b, 0)),
            pl.BlockSpec((1, DH, 128), lambda b, t: (t, 0, 0)),
            pl.BlockSpec((1, 128), lambda b, t: (0, 0)),
        ],
        out_specs=pl.BlockSpec((NB, 128), lambda b, t: (b, 0)),
        out_shape=jax.ShapeDtypeStruct((NPAD, 128), jnp.float32),
        scratch_shapes=[
            pltpu.VMEM((NB, DH), jnp.float32),
            pltpu.VMEM((NB, 128), jnp.float32),
        ],
    )(yout, wo, bo)

    return out[:N, :DOUT]
